# Initial kernel scaffold; baseline (speedup 1.0000x reference)
import jax
import jax.numpy as jnp
from jax.experimental import pallas as pl


def kernel(x, conv_w, bn_w, bn_b):
    raise NotImplementedError("write your pallas kernel here")


# speedup vs baseline: 2.9155x; 2.9155x over previous
import jax
import jax.numpy as jnp
from jax.experimental import pallas as pl

_H = 32
_W = 32
_N = _H * _W


def _vshift(a, dy, lane):
    if dy == 0:
        return a
    if dy == 1:
        return jnp.where(lane >= _N - _W, a, jnp.roll(a, -_W, axis=1))
    return jnp.where(lane < _W, a, jnp.roll(a, _W, axis=1))


def _hshift(a, dx, xcol):
    if dx == 0:
        return a
    if dx == 1:
        return jnp.where(xcol == _W - 1, a, jnp.roll(a, -1, axis=1))
    return jnp.where(xcol == 0, a, jnp.roll(a, 1, axis=1))


def _normalize_cols(v):
    n = jnp.sqrt(jnp.sum(v * v, axis=0, keepdims=True))
    return v / jnp.maximum(n, 1e-12)


def _nn_conv_body(prev_ref, next_ref, cur_ref, w_ref, out_ref, s1_ref, s2_ref):
    prev = prev_ref[0]
    nxt = next_ref[0]
    cur = cur_ref[0]

    lane = jax.lax.broadcasted_iota(jnp.int32, (1, _N), 1)
    xcol = lane & (_W - 1)

    cur_n = _normalize_cols(cur)

    def find_nn(y):
        yk = _normalize_cols(y)
        sims = []
        for dy in (-1, 0, 1):
            v = _vshift(yk, dy, lane)
            for dx in (-1, 0, 1):
                hs = _hshift(v, dx, xcol)
                sims.append(jnp.sum(cur_n * hs, axis=0, keepdims=True))
        best = sims[0]
        bestd = jnp.zeros((1, _N), jnp.int32)
        for d in range(1, 9):
            better = sims[d] < best
            best = jnp.where(better, sims[d], best)
            bestd = jnp.where(better, d, bestd)
        nn = jnp.zeros_like(yk)
        d = 0
        for dy in (-1, 0, 1):
            v = _vshift(yk, dy, lane)
            for dx in (-1, 0, 1):
                hs = _hshift(v, dx, xcol)
                nn = jnp.where(bestd == d, hs, nn)
                d += 1
        return nn

    nn_prev = find_nn(prev)
    nn_next = find_nn(nxt)
    cat = jnp.concatenate([nn_prev, nn_next, cur], axis=0)
    out = jax.lax.dot_general(
        w_ref[...], cat, (((1,), (0,)), ((), ())),
        preferred_element_type=jnp.float32,
        precision=jax.lax.Precision.HIGHEST)
    out_ref[0] = out
    s1_ref[0] = jnp.sum(out, axis=1, keepdims=True)
    s2_ref[0] = jnp.sum(out * out, axis=1, keepdims=True)


def _bn_body(out_ref, s1_ref, s2_ref, w_ref, b_ref, y_ref):
    cnt = 8.0 * _N
    s1 = jnp.sum(s1_ref[...], axis=0)
    s2 = jnp.sum(s2_ref[...], axis=0)
    mean = s1 / cnt
    var = s2 / cnt - mean * mean
    inv = w_ref[...] / jnp.sqrt(var + 1e-5)
    y = (out_ref[0] - mean) * inv + b_ref[...]
    y_ref[0] = jnp.maximum(y, 0.0)


def kernel(x, conv_w, bn_w, bn_b):
    b, _, c, h, w = x.shape
    n = h * w
    prev = x[:, 0].reshape(b, c, n)
    nxt = x[:, 1].reshape(b, c, n)
    cur = x[:, 2].reshape(b, c, n)

    out_pre, s1, s2 = pl.pallas_call(
        _nn_conv_body,
        grid=(b,),
        in_specs=[
            pl.BlockSpec((1, c, n), lambda i: (i, 0, 0)),
            pl.BlockSpec((1, c, n), lambda i: (i, 0, 0)),
            pl.BlockSpec((1, c, n), lambda i: (i, 0, 0)),
            pl.BlockSpec((c, 3 * c), lambda i: (0, 0)),
        ],
        out_specs=[
            pl.BlockSpec((1, c, n), lambda i: (i, 0, 0)),
            pl.BlockSpec((1, c, 1), lambda i: (i, 0, 0)),
            pl.BlockSpec((1, c, 1), lambda i: (i, 0, 0)),
        ],
        out_shape=[
            jax.ShapeDtypeStruct((b, c, n), jnp.float32),
            jax.ShapeDtypeStruct((b, c, 1), jnp.float32),
            jax.ShapeDtypeStruct((b, c, 1), jnp.float32),
        ],
    )(prev, nxt, cur, conv_w)

    y = pl.pallas_call(
        _bn_body,
        grid=(b,),
        in_specs=[
            pl.BlockSpec((1, c, n), lambda i: (i, 0, 0)),
            pl.BlockSpec((b, c, 1), lambda i: (0, 0, 0)),
            pl.BlockSpec((b, c, 1), lambda i: (0, 0, 0)),
            pl.BlockSpec((c, 1), lambda i: (0, 0)),
            pl.BlockSpec((c, 1), lambda i: (0, 0)),
        ],
        out_specs=pl.BlockSpec((1, c, n), lambda i: (i, 0, 0)),
        out_shape=jax.ShapeDtypeStruct((b, c, n), jnp.float32),
    )(out_pre, s1, s2, bn_w.reshape(c, 1), bn_b.reshape(c, 1))

    return jnp.stack([x[:, 0], x[:, 1], y.reshape(b, c, h, w)], axis=1)


# speedup vs baseline: 4.0303x; 1.3824x over previous
import jax
import jax.numpy as jnp
from jax.experimental import pallas as pl
from jax.experimental.pallas import tpu as pltpu

_H = 32
_W = 32
_N = _H * _W
_B = 8
_C = 192


def _vshift(a, dy, lane):
    if dy == 0:
        return a
    if dy == 1:
        return jnp.where(lane >= _N - _W, a, jnp.roll(a, -_W, axis=1))
    return jnp.where(lane < _W, a, jnp.roll(a, _W, axis=1))


def _hshift(a, dx, xcol):
    if dx == 0:
        return a
    if dx == 1:
        return jnp.where(xcol == _W - 1, a, jnp.roll(a, -1, axis=1))
    return jnp.where(xcol == 0, a, jnp.roll(a, 1, axis=1))


def _normalize_cols(v):
    n = jnp.sqrt(jnp.sum(v * v, axis=0, keepdims=True))
    return v / jnp.maximum(n, 1e-12)


def _body(x_ref, w_ref, bnw_ref, bnb_ref, y_ref, pre_scr, pn_scr, s1_scr, s2_scr):
    i = pl.program_id(0)

    @pl.when(i < _B)
    def _phase1():
        prev = x_ref[0, 0]
        nxt = x_ref[0, 1]
        cur = x_ref[0, 2]

        lane = jax.lax.broadcasted_iota(jnp.int32, (1, _N), 1)
        xcol = lane & (_W - 1)
        cur_n = _normalize_cols(cur)

        def find_nn(y):
            yk = _normalize_cols(y)
            sims = []
            for dy in (-1, 0, 1):
                v = _vshift(yk, dy, lane)
                for dx in (-1, 0, 1):
                    hs = _hshift(v, dx, xcol)
                    sims.append(jnp.sum(cur_n * hs, axis=0, keepdims=True))
            best = sims[0]
            bestd = jnp.zeros((1, _N), jnp.int32)
            for d in range(1, 9):
                better = sims[d] < best
                best = jnp.where(better, sims[d], best)
                bestd = jnp.where(better, d, bestd)
            nn = jnp.zeros_like(yk)
            d = 0
            for dy in (-1, 0, 1):
                v = _vshift(yk, dy, lane)
                for dx in (-1, 0, 1):
                    hs = _hshift(v, dx, xcol)
                    nn = jnp.where(bestd == d, hs, nn)
                    d += 1
            return nn

        nn_prev = find_nn(prev)
        nn_next = find_nn(nxt)
        cat = jnp.concatenate([nn_prev, nn_next, cur], axis=0)
        out = jax.lax.dot_general(
            w_ref[...], cat, (((1,), (0,)), ((), ())),
            preferred_element_type=jnp.float32,
            precision=jax.lax.Precision.HIGHEST)
        pre_scr[i] = out
        pn_scr[i] = x_ref[0, :2]
        p1 = jnp.sum(out, axis=1, keepdims=True)
        p2 = jnp.sum(out * out, axis=1, keepdims=True)
        is0 = i == 0
        s1_scr[...] = jnp.where(is0, p1, s1_scr[...] + p1)
        s2_scr[...] = jnp.where(is0, p2, s2_scr[...] + p2)

    @pl.when(i >= _B)
    def _phase2():
        b = i - _B
        cnt = float(_B * _N)
        mean = s1_scr[...] / cnt
        var = s2_scr[...] / cnt - mean * mean
        inv = bnw_ref[...] / jnp.sqrt(var + 1e-5)
        y_ref[0, :2] = pn_scr[b]
        y_ref[0, 2] = jnp.maximum((pre_scr[b] - mean) * inv + bnb_ref[...], 0.0)


def kernel(x, conv_w, bn_w, bn_b):
    b, f, c, h, w = x.shape
    n = h * w
    x4 = x.reshape(b, f, c, n)

    y = pl.pallas_call(
        _body,
        grid=(2 * b,),
        in_specs=[
            pl.BlockSpec((1, f, c, n), lambda i: (jnp.minimum(i, _B - 1), 0, 0, 0)),
            pl.BlockSpec((c, f * c), lambda i: (0, 0)),
            pl.BlockSpec((c, 1), lambda i: (0, 0)),
            pl.BlockSpec((c, 1), lambda i: (0, 0)),
        ],
        out_specs=pl.BlockSpec(
            (1, f, c, n), lambda i: (jnp.maximum(i - _B, 0), 0, 0, 0)),
        out_shape=jax.ShapeDtypeStruct((b, f, c, n), jnp.float32),
        scratch_shapes=[
            pltpu.VMEM((_B, _C, _N), jnp.float32),
            pltpu.VMEM((_B, 2, _C, _N), jnp.float32),
            pltpu.VMEM((_C, 1), jnp.float32),
            pltpu.VMEM((_C, 1), jnp.float32),
        ],
    )(x4, conv_w, bn_w.reshape(c, 1), bn_b.reshape(c, 1))

    return y.reshape(b, f, c, h, w)


# speedup vs baseline: 4.0622x; 1.0079x over previous
import jax
import jax.numpy as jnp
from jax.experimental import pallas as pl
from jax.experimental.pallas import tpu as pltpu

_H = 32
_W = 32
_N = _H * _W
_B = 8
_C = 192


def _vshift(a, dy, lane):
    if dy == 0:
        return a
    if dy == 1:
        return jnp.where(lane >= _N - _W, a, jnp.roll(a, -_W, axis=1))
    return jnp.where(lane < _W, a, jnp.roll(a, _W, axis=1))


def _hshift(a, dx, xcol):
    if dx == 0:
        return a
    if dx == 1:
        return jnp.where(xcol == _W - 1, a, jnp.roll(a, -1, axis=1))
    return jnp.where(xcol == 0, a, jnp.roll(a, 1, axis=1))


def _normalize_cols(v):
    n = jnp.sqrt(jnp.sum(v * v, axis=0, keepdims=True))
    return v / jnp.maximum(n, 1e-12)


def _body(x_ref, w_ref, bnw_ref, bnb_ref, y_ref, pre_scr, pn_scr, s1_scr, s2_scr):
    i = pl.program_id(0)

    @pl.when(i < _B)
    def _phase1():
        prev = x_ref[0, 0]
        nxt = x_ref[0, 1]
        cur = x_ref[0, 2]

        lane = jax.lax.broadcasted_iota(jnp.int32, (1, _N), 1)
        xcol = lane & (_W - 1)
        cur_n = _normalize_cols(cur)

        def find_nn(y):
            yk = _normalize_cols(y)
            best = None
            nn = None
            for dy in (-1, 0, 1):
                v = _vshift(yk, dy, lane)
                for dx in (-1, 0, 1):
                    hs = _hshift(v, dx, xcol)
                    s = jnp.sum(cur_n * hs, axis=0, keepdims=True)
                    if best is None:
                        best, nn = s, hs
                    else:
                        better = s < best
                        best = jnp.where(better, s, best)
                        nn = jnp.where(better, hs, nn)
            return nn

        nn_prev = find_nn(prev)
        nn_next = find_nn(nxt)
        cat = jnp.concatenate([nn_prev, nn_next, cur], axis=0)
        out = jax.lax.dot_general(
            w_ref[...], cat, (((1,), (0,)), ((), ())),
            preferred_element_type=jnp.float32,
            precision=jax.lax.Precision.HIGHEST)
        pre_scr[i] = out
        pn_scr[i] = x_ref[0, :2]
        p1 = jnp.sum(out, axis=1, keepdims=True)
        p2 = jnp.sum(out * out, axis=1, keepdims=True)
        is0 = i == 0
        s1_scr[...] = jnp.where(is0, p1, s1_scr[...] + p1)
        s2_scr[...] = jnp.where(is0, p2, s2_scr[...] + p2)

    @pl.when(i >= _B)
    def _phase2():
        b = i - _B
        cnt = float(_B * _N)
        mean = s1_scr[...] / cnt
        var = s2_scr[...] / cnt - mean * mean
        inv = bnw_ref[...] / jnp.sqrt(var + 1e-5)
        y_ref[0, :2] = pn_scr[b]
        y_ref[0, 2] = jnp.maximum((pre_scr[b] - mean) * inv + bnb_ref[...], 0.0)


def kernel(x, conv_w, bn_w, bn_b):
    b, f, c, h, w = x.shape
    n = h * w
    x4 = x.reshape(b, f, c, n)

    y = pl.pallas_call(
        _body,
        grid=(2 * b,),
        in_specs=[
            pl.BlockSpec((1, f, c, n), lambda i: (jnp.minimum(i, _B - 1), 0, 0, 0)),
            pl.BlockSpec((c, f * c), lambda i: (0, 0)),
            pl.BlockSpec((c, 1), lambda i: (0, 0)),
            pl.BlockSpec((c, 1), lambda i: (0, 0)),
        ],
        out_specs=pl.BlockSpec(
            (1, f, c, n), lambda i: (jnp.maximum(i - _B, 0), 0, 0, 0)),
        out_shape=jax.ShapeDtypeStruct((b, f, c, n), jnp.float32),
        scratch_shapes=[
            pltpu.VMEM((_B, _C, _N), jnp.float32),
            pltpu.VMEM((_B, 2, _C, _N), jnp.float32),
            pltpu.VMEM((_C, 1), jnp.float32),
            pltpu.VMEM((_C, 1), jnp.float32),
        ],
    )(x4, conv_w, bn_w.reshape(c, 1), bn_b.reshape(c, 1))

    return y.reshape(b, f, c, h, w)


# speedup vs baseline: 4.0651x; 1.0007x over previous
import jax
import jax.numpy as jnp
from jax.experimental import pallas as pl
from jax.experimental.pallas import tpu as pltpu

_H = 32
_W = 32
_N = _H * _W
_B = 8
_C = 192


def _vshift(a, dy, lane):
    if dy == 0:
        return a
    if dy == 1:
        return jnp.where(lane >= _N - _W, a, jnp.roll(a, -_W, axis=1))
    return jnp.where(lane < _W, a, jnp.roll(a, _W, axis=1))


def _hshift(a, dx, xcol):
    if dx == 0:
        return a
    if dx == 1:
        return jnp.where(xcol == _W - 1, a, jnp.roll(a, -1, axis=1))
    return jnp.where(xcol == 0, a, jnp.roll(a, 1, axis=1))


def _normalize_cols(v):
    n = jnp.sqrt(jnp.sum(v * v, axis=0, keepdims=True))
    return v / jnp.maximum(n, 1e-12)


def _body(x_ref, w_ref, bnw_ref, bnb_ref, y_ref, pre_scr, pn_scr, s1_scr, s2_scr):
    i = pl.program_id(0)

    @pl.when(i < _B)
    def _phase1():
        prev = x_ref[0, 0]
        nxt = x_ref[0, 1]
        cur = x_ref[0, 2]

        lane = jax.lax.broadcasted_iota(jnp.int32, (1, _N), 1)
        xcol = lane & (_W - 1)
        xm_m = xcol == 0
        xm_p = xcol == _W - 1
        ym_m = lane < _W
        ym_p = lane >= _N - _W
        cur_n = _normalize_cols(cur)

        def find_nn(y):
            yk = _normalize_cols(y)
            hs = {}
            sraw = {}
            for dy in (-1, 0, 1):
                v = yk if dy == 0 else jnp.roll(yk, -_W * dy, axis=1)
                for dx in (-1, 0, 1):
                    m = v if dx == 0 else jnp.roll(v, -dx, axis=1)
                    hs[(dy, dx)] = m
                    sraw[(dy, dx)] = jnp.sum(cur_n * m, axis=0, keepdims=True)
            sx = {}
            for dy in (-1, 0, 1):
                sx[(dy, -1)] = jnp.where(xm_m, sraw[(dy, 0)], sraw[(dy, -1)])
                sx[(dy, 0)] = sraw[(dy, 0)]
                sx[(dy, 1)] = jnp.where(xm_p, sraw[(dy, 0)], sraw[(dy, 1)])
            best = None
            bestd = None
            d = 0
            for dy in (-1, 0, 1):
                for dx in (-1, 0, 1):
                    if dy == -1:
                        s = jnp.where(ym_m, sx[(0, dx)], sx[(-1, dx)])
                    elif dy == 1:
                        s = jnp.where(ym_p, sx[(0, dx)], sx[(1, dx)])
                    else:
                        s = sx[(0, dx)]
                    if best is None:
                        best = s
                        bestd = jnp.zeros((1, _N), jnp.int32)
                    else:
                        better = s < best
                        best = jnp.where(better, s, best)
                        bestd = jnp.where(better, d, bestd)
                    d += 1
            dyq = bestd // 3
            dxq = bestd - dyq * 3
            dyc = jnp.where(ym_m, jnp.maximum(dyq, 1), dyq)
            dyc = jnp.where(ym_p, jnp.minimum(dyc, 1), dyc)
            dxc = jnp.where(xm_m, jnp.maximum(dxq, 1), dxq)
            dxc = jnp.where(xm_p, jnp.minimum(dxc, 1), dxc)
            e = dyc * 3 + dxc
            nn = hs[(-1, -1)]
            d = 0
            for dy in (-1, 0, 1):
                for dx in (-1, 0, 1):
                    if d > 0:
                        nn = jnp.where(e == d, hs[(dy, dx)], nn)
                    d += 1
            return nn

        nn_prev = find_nn(prev)
        nn_next = find_nn(nxt)
        cat = jnp.concatenate([nn_prev, nn_next, cur], axis=0)
        out = jax.lax.dot_general(
            w_ref[...], cat, (((1,), (0,)), ((), ())),
            preferred_element_type=jnp.float32,
            precision=jax.lax.Precision.HIGHEST)
        pre_scr[i] = out
        pn_scr[i] = x_ref[0, :2]
        p1 = jnp.sum(out, axis=1, keepdims=True)
        p2 = jnp.sum(out * out, axis=1, keepdims=True)
        is0 = i == 0
        s1_scr[...] = jnp.where(is0, p1, s1_scr[...] + p1)
        s2_scr[...] = jnp.where(is0, p2, s2_scr[...] + p2)

    @pl.when(i >= _B)
    def _phase2():
        b = i - _B
        cnt = float(_B * _N)
        mean = s1_scr[...] / cnt
        var = s2_scr[...] / cnt - mean * mean
        inv = bnw_ref[...] / jnp.sqrt(var + 1e-5)
        y_ref[0, :2] = pn_scr[b]
        y_ref[0, 2] = jnp.maximum((pre_scr[b] - mean) * inv + bnb_ref[...], 0.0)


def kernel(x, conv_w, bn_w, bn_b):
    b, f, c, h, w = x.shape
    n = h * w
    x4 = x.reshape(b, f, c, n)

    y = pl.pallas_call(
        _body,
        grid=(2 * b,),
        in_specs=[
            pl.BlockSpec((1, f, c, n), lambda i: (jnp.minimum(i, _B - 1), 0, 0, 0)),
            pl.BlockSpec((c, f * c), lambda i: (0, 0)),
            pl.BlockSpec((c, 1), lambda i: (0, 0)),
            pl.BlockSpec((c, 1), lambda i: (0, 0)),
        ],
        out_specs=pl.BlockSpec(
            (1, f, c, n), lambda i: (jnp.maximum(i - _B, 0), 0, 0, 0)),
        out_shape=jax.ShapeDtypeStruct((b, f, c, n), jnp.float32),
        scratch_shapes=[
            pltpu.VMEM((_B, _C, _N), jnp.float32),
            pltpu.VMEM((_B, 2, _C, _N), jnp.float32),
            pltpu.VMEM((_C, 1), jnp.float32),
            pltpu.VMEM((_C, 1), jnp.float32),
        ],
    )(x4, conv_w, bn_w.reshape(c, 1), bn_b.reshape(c, 1))

    return y.reshape(b, f, c, h, w)
